# baseline (speedup 1.0000x reference)
import functools

import jax
import jax.numpy as jnp
from jax import lax
from jax.experimental import pallas as pl
from jax.experimental.pallas import tpu as pltpu
from jax.experimental.pallas import tpu_sc as plsc

NUM_EMB = 100000
D = 128
B = 16384
H = 50
HP = 56
NC = 2
NS = 16
NW = NC * NS
PHASES = 2
BP = B // PHASES
SAMP_PER_W = BP // NW
PAIRS_PER_W = SAMP_PER_W // 2
IDX_PER_W = SAMP_PER_W * HP
NBUF = 2
GROUPS = PAIRS_PER_W // NBUF


def _emb_body(x_hbm, tab_hbm, out, idx_v, buf, gsem, wsem):
    wid = lax.axis_index("s") * NC + lax.axis_index("c")
    base_s = wid * SAMP_PER_W
    pltpu.sync_copy(x_hbm.at[pl.ds(wid * IDX_PER_W, IDX_PER_W)], idx_v)

    def gather(p, b):
        sl = idx_v.at[pl.ds(p * (2 * HP), 2 * HP)]
        pltpu.async_copy(tab_hbm.at[sl], buf.at[b], gsem.at[b])

    def write(p, b):
        s0 = base_s + 2 * p
        pltpu.async_copy(buf.at[b, pl.ds(0, H)], out.at[s0], wsem.at[b])
        pltpu.async_copy(buf.at[b, pl.ds(HP, H)], out.at[s0 + 1], wsem.at[b])

    def wait_writes(b):
        pltpu.make_async_copy(buf.at[b, pl.ds(0, H)], out.at[base_s],
                              wsem.at[b]).wait()
        pltpu.make_async_copy(buf.at[b, pl.ds(HP, H)], out.at[base_s],
                              wsem.at[b]).wait()

    for b in range(NBUF):
        gather(b, b)

    def body(g, carry):
        base = g * NBUF
        for b in range(NBUF):
            pltpu.make_async_copy(tab_hbm.at[idx_v.at[pl.ds(0, 2 * HP)]],
                                  buf.at[b], gsem.at[b]).wait()
            write(base + b, b)

        @pl.when(g < GROUPS - 1)
        def _():
            for b in range(NBUF):
                wait_writes(b)
                gather(base + NBUF + b, b)

        return carry

    lax.fori_loop(0, GROUPS, body, 0)

    for b in range(NBUF):
        wait_writes(b)


@jax.jit
def _run(x1d, real_w, imag_w):
    mesh = plsc.VectorSubcoreMesh(core_axis_name="c", subcore_axis_name="s")
    f = functools.partial(
        pl.kernel,
        out_type=jax.ShapeDtypeStruct((BP, H, D), jnp.float32),
        mesh=mesh,
        scratch_types=[
            pltpu.VMEM((IDX_PER_W,), jnp.int32),
            pltpu.VMEM((NBUF, 2 * HP, D), jnp.float32),
            pltpu.SemaphoreType.DMA((NBUF,)),
            pltpu.SemaphoreType.DMA((NBUF,)),
        ],
    )(_emb_body)
    seg = BP * HP
    parts = []
    for tab in (real_w, imag_w):
        for ph in range(PHASES):
            parts.append(f(lax.slice(x1d, (ph * seg,), ((ph + 1) * seg,)),
                           tab))
    real = jnp.concatenate(parts[:PHASES], axis=0)
    imag = jnp.concatenate(parts[PHASES:], axis=0)
    return real, imag


def kernel(x, real_w, imag_w):
    xi = x.astype(jnp.int32)
    xp = jnp.concatenate([xi, xi[:, : HP - H]], axis=1)
    return _run(xp.reshape(B * HP), real_w, imag_w)


# speedup vs baseline: 1.5409x; 1.5409x over previous
import functools

import jax
import jax.numpy as jnp
from jax import lax
from jax.experimental import pallas as pl
from jax.experimental.pallas import tpu as pltpu
from jax.experimental.pallas import tpu_sc as plsc

NUM_EMB = 100000
D = 128
B = 16384
H = 50
HP = 56
NC = 2
NS = 16
NW = NC * NS
BP = B
SAMP_PER_W = BP // NW
PAIRS_PER_W = SAMP_PER_W // 2
IDX_PER_W = SAMP_PER_W * HP
NBUF = 2
GROUPS = PAIRS_PER_W // NBUF


def _emb_body(x_hbm, tab_hbm, out, idx_v, buf, gsem, wsem):
    wid = lax.axis_index("s") * NC + lax.axis_index("c")
    base_s = wid * SAMP_PER_W
    pltpu.sync_copy(x_hbm.at[pl.ds(wid * IDX_PER_W, IDX_PER_W)], idx_v)

    def gather(p, b):
        sl = idx_v.at[pl.ds(p * (2 * HP), 2 * HP)]
        pltpu.async_copy(tab_hbm.at[sl], buf.at[b], gsem.at[b])

    def write(p, b):
        s0 = base_s + 2 * p
        pltpu.async_copy(buf.at[b, pl.ds(0, H)], out.at[s0], wsem.at[b])
        pltpu.async_copy(buf.at[b, pl.ds(HP, H)], out.at[s0 + 1], wsem.at[b])

    def wait_writes(b):
        pltpu.make_async_copy(buf.at[b, pl.ds(0, H)], out.at[base_s],
                              wsem.at[b]).wait()
        pltpu.make_async_copy(buf.at[b, pl.ds(HP, H)], out.at[base_s],
                              wsem.at[b]).wait()

    for b in range(NBUF):
        gather(b, b)

    def body(g, carry):
        base = g * NBUF
        for b in range(NBUF):
            pltpu.make_async_copy(tab_hbm.at[idx_v.at[pl.ds(0, 2 * HP)]],
                                  buf.at[b], gsem.at[b]).wait()
            write(base + b, b)

        @pl.when(g < GROUPS - 1)
        def _():
            for b in range(NBUF):
                wait_writes(b)
                gather(base + NBUF + b, b)

        return carry

    lax.fori_loop(0, GROUPS, body, 0)

    for b in range(NBUF):
        wait_writes(b)


@jax.jit
def _run(x1d, real_w, imag_w):
    mesh = plsc.VectorSubcoreMesh(core_axis_name="c", subcore_axis_name="s")
    f = functools.partial(
        pl.kernel,
        out_type=jax.ShapeDtypeStruct((BP, H, D), jnp.float32),
        mesh=mesh,
        scratch_types=[
            pltpu.VMEM((IDX_PER_W,), jnp.int32),
            pltpu.VMEM((NBUF, 2 * HP, D), jnp.float32),
            pltpu.SemaphoreType.DMA((NBUF,)),
            pltpu.SemaphoreType.DMA((NBUF,)),
        ],
    )(_emb_body)
    real = f(x1d, real_w)
    imag = f(x1d, imag_w)
    return real, imag


def kernel(x, real_w, imag_w):
    xi = x.astype(jnp.int32)
    xp = jnp.concatenate([xi, xi[:, : HP - H]], axis=1)
    return _run(xp.reshape(B * HP), real_w, imag_w)
